# baseline (speedup 1.0000x reference)
import functools

import jax
import jax.numpy as jnp
from jax.experimental import pallas as pl
from jax.experimental.pallas import tpu as pltpu
from jax.experimental.pallas import tpu_sc as plsc

_B = 16384
_D = 32
_F = 26
_SPARSE_V = 100000
_DIN = (_F + 2) * _D
_H = 2 * _D
_W = 128
_BB = 512
_NJ = _F + 2


@functools.cache
def _sc_gather_kernel():
    mesh = plsc.VectorSubcoreMesh(core_axis_name="c", subcore_axis_name="s")

    @functools.partial(
        pl.kernel,
        out_type=jax.ShapeDtypeStruct((_NJ * _B, 128), jnp.float32),
        mesh=mesh,
        compiler_params=pltpu.CompilerParams(use_tc_tiling_on_sc=False),
    )
    def sc_gather(u_hbm, i_hbm, s_hbm, ui_hbm, ii_hbm, si_hbm, out_hbm):
        def mk(tbl):
            def body(i_vmem, o_vmem):
                pltpu.sync_copy(tbl.at[i_vmem.at[0]], o_vmem)
            return body

        item_base = _B // _W
        sp_base = 2 * _B // _W
        pltpu.emit_pipeline(
            mk(u_hbm),
            grid=(_B // _W,),
            in_specs=[pl.BlockSpec((1, _W), lambda i: (0, i))],
            out_specs=[pl.BlockSpec((_W, 128), lambda i: (i, 0))],
            core_axis_name=("c", "s"),
            dimension_semantics=(pltpu.PARALLEL,),
        )(ui_hbm, out_hbm)
        pltpu.emit_pipeline(
            mk(i_hbm),
            grid=(_B // _W,),
            in_specs=[pl.BlockSpec((1, _W), lambda i: (0, i))],
            out_specs=[pl.BlockSpec((_W, 128), lambda i: (item_base + i, 0))],
            core_axis_name=("c", "s"),
            dimension_semantics=(pltpu.PARALLEL,),
        )(ii_hbm, out_hbm)
        pltpu.emit_pipeline(
            mk(s_hbm),
            grid=(_F * _B // _W,),
            in_specs=[pl.BlockSpec((1, _W), lambda i: (0, i))],
            out_specs=[pl.BlockSpec((_W, 128), lambda i: (sp_base + i, 0))],
            core_axis_name=("c", "s"),
            dimension_semantics=(pltpu.PARALLEL,),
        )(si_hbm, out_hbm)

    return sc_gather


def _dense_body(xall_ref, sel_ref, wW_ref, wb_ref, w0_ref, b0_ref,
                w1_ref, b1_ref, w2_ref, b2_ref, w3_ref, b3_ref,
                tw_ref, tb_ref, o_ref):
    dot = lambda a, b: jax.lax.dot_general(
        a, b, (((1,), (0,)), ((), ())), preferred_element_type=jnp.float32)
    w0 = w0_ref[...]
    wW = wW_ref[...]
    hacc = None
    wacc = None
    for j in range(_NJ):
        x4 = xall_ref[j]
        sel = sel_ref[:, j:j + 1]
        e = jnp.where(sel == 0, x4[:, 0:32],
            jnp.where(sel == 1, x4[:, 32:64],
            jnp.where(sel == 2, x4[:, 64:96], x4[:, 96:128])))
        hj = dot(e, w0[32 * j:32 * j + 32])
        wj = dot(e, wW[32 * j:32 * j + 32])
        hacc = hj if hacc is None else hacc + hj
        wacc = wj if wacc is None else wacc + wj
    h = jax.nn.relu(hacc + b0_ref[...])
    h = jax.nn.relu(dot(h, w1_ref[...]) + b1_ref[...])
    h = jax.nn.relu(dot(h, w2_ref[...]) + b2_ref[...])
    deep = dot(h, w3_ref[...]) + b3_ref[...]
    wide = wacc + wb_ref[...]
    tw = tw_ref[...]
    logit = (jnp.sum(wide * tw[:, 0:_D], axis=1, keepdims=True)
             + jnp.sum(deep * tw[:, _D:], axis=1, keepdims=True)
             + tb_ref[...])
    o_ref[...] = jax.nn.sigmoid(logit)


def _dense_forward(xall, sel, wide_W, wide_b, dW0, db0, dW1, db1,
                   dW2, db2, dW3, db3, tW, tb):
    row = lambda i: (i, 0)
    full = lambda i: (0, 0)
    return pl.pallas_call(
        _dense_body,
        grid=(_B // _BB,),
        in_specs=[
            pl.BlockSpec((_NJ, _BB, 128), lambda i: (0, i, 0)),
            pl.BlockSpec((_BB, _NJ), row),
            pl.BlockSpec((_DIN, _D), full),
            pl.BlockSpec((1, _D), full),
            pl.BlockSpec((_DIN, _H), full),
            pl.BlockSpec((1, _H), full),
            pl.BlockSpec((_H, _H), full),
            pl.BlockSpec((1, _H), full),
            pl.BlockSpec((_H, _H), full),
            pl.BlockSpec((1, _H), full),
            pl.BlockSpec((_H, _D), full),
            pl.BlockSpec((1, _D), full),
            pl.BlockSpec((1, 2 * _D), full),
            pl.BlockSpec((1, 1), full),
        ],
        out_specs=pl.BlockSpec((_BB, 1), row),
        out_shape=jax.ShapeDtypeStruct((_B, 1), jnp.float32),
    )(xall, sel, wide_W, wide_b.reshape(1, _D), dW0, db0.reshape(1, _H),
      dW1, db1.reshape(1, _H), dW2, db2.reshape(1, _H), dW3,
      db3.reshape(1, _D), tW.reshape(1, 2 * _D), tb.reshape(1, 1))


def kernel(user_ids, item_ids, sparse_features, user_table, item_table,
           sparse_tables, wide_W, wide_b, dW0, db0, dW1, db1, dW2, db2,
           dW3, db3, tW, tb):
    u4 = user_table.reshape(-1, 128)
    i4 = item_table.reshape(-1, 128)
    s4 = sparse_tables.reshape(-1, 128)
    ui = (user_ids // 4).reshape(1, _B)
    ii = (item_ids // 4).reshape(1, _B)
    sf_t = sparse_features.T
    sp_offs = (jnp.arange(_F, dtype=jnp.int32) * (_SPARSE_V // 4))[:, None]
    si = (sf_t // 4 + sp_offs).reshape(1, _F * _B)
    sel = jnp.concatenate([(user_ids % 4)[:, None], (item_ids % 4)[:, None],
                           sparse_features % 4], axis=1)
    rows = _sc_gather_kernel()(u4, i4, s4, ui, ii, si)
    xall = rows.reshape(_NJ, _B, 128)
    return _dense_forward(xall, sel, wide_W, wide_b, dW0, db0,
                          dW1, db1, dW2, db2, dW3, db3, tW, tb)


# speedup vs baseline: 1.1183x; 1.1183x over previous
import functools

import jax
import jax.numpy as jnp
from jax.experimental import pallas as pl
from jax.experimental.pallas import tpu as pltpu
from jax.experimental.pallas import tpu_sc as plsc

_B = 16384
_D = 32
_F = 26
_SPARSE_V = 100000
_DIN = (_F + 2) * _D
_H = 2 * _D
_W = 128
_BB = 512
_NJ = _F + 2


def _gather_pipeline(table_hbm, idx_hbm, idx_row, out_hbm):
    def body(i_vmem, o_vmem):
        pltpu.sync_copy(table_hbm.at[i_vmem.at[0]], o_vmem)

    pltpu.emit_pipeline(
        body,
        grid=(_B // _W,),
        in_specs=[pl.BlockSpec((1, _W), lambda i, r=idx_row: (r, i))],
        out_specs=[pl.BlockSpec((_W, _D), lambda i: (i, 0))],
        core_axis_name=("c", "s"),
        dimension_semantics=(pltpu.PARALLEL,),
    )(idx_hbm, out_hbm)


@functools.cache
def _sc_gather_kernel():
    mesh = plsc.VectorSubcoreMesh(core_axis_name="c", subcore_axis_name="s")
    emb = jax.ShapeDtypeStruct((_B, _D), jnp.float32)

    @functools.partial(
        pl.kernel,
        out_type=(emb,) * _NJ,
        mesh=mesh,
        compiler_params=pltpu.CompilerParams(use_tc_tiling_on_sc=False),
    )
    def sc_gather(u_hbm, i_hbm, s_hbm, ui_hbm, ii_hbm, sf_hbm, *outs):
        _gather_pipeline(u_hbm, ui_hbm, 0, outs[0])
        _gather_pipeline(i_hbm, ii_hbm, 0, outs[1])
        for f in range(_F):
            _gather_pipeline(s_hbm.at[f], sf_hbm, f, outs[2 + f])

    return sc_gather


def _dense_body(*refs):
    x_refs = refs[:_NJ]
    (wW_ref, wb_ref, w0_ref, b0_ref, w1_ref, b1_ref, w2_ref, b2_ref,
     w3_ref, b3_ref, tw_ref, tb_ref, o_ref) = refs[_NJ:]
    dot = lambda a, b: jax.lax.dot_general(
        a, b, (((1,), (0,)), ((), ())), preferred_element_type=jnp.float32)
    w0 = w0_ref[...]
    wW = wW_ref[...]
    hacc = None
    wacc = None
    for j in range(_NJ):
        e = x_refs[j][...]
        hj = dot(e, w0[32 * j:32 * j + 32])
        wj = dot(e, wW[32 * j:32 * j + 32])
        hacc = hj if hacc is None else hacc + hj
        wacc = wj if wacc is None else wacc + wj
    h = jax.nn.relu(hacc + b0_ref[...])
    h = jax.nn.relu(dot(h, w1_ref[...]) + b1_ref[...])
    h = jax.nn.relu(dot(h, w2_ref[...]) + b2_ref[...])
    deep = dot(h, w3_ref[...]) + b3_ref[...]
    wide = wacc + wb_ref[...]
    tw = tw_ref[...]
    logit = (jnp.sum(wide * tw[:, 0:_D], axis=1, keepdims=True)
             + jnp.sum(deep * tw[:, _D:], axis=1, keepdims=True)
             + tb_ref[...])
    o_ref[...] = jax.nn.sigmoid(logit)


def _dense_forward(xs, wide_W, wide_b, dW0, db0, dW1, db1,
                   dW2, db2, dW3, db3, tW, tb):
    row = lambda i: (i, 0)
    full = lambda i: (0, 0)
    return pl.pallas_call(
        _dense_body,
        grid=(_B // _BB,),
        in_specs=[pl.BlockSpec((_BB, _D), row)] * _NJ + [
            pl.BlockSpec((_DIN, _D), full),
            pl.BlockSpec((1, _D), full),
            pl.BlockSpec((_DIN, _H), full),
            pl.BlockSpec((1, _H), full),
            pl.BlockSpec((_H, _H), full),
            pl.BlockSpec((1, _H), full),
            pl.BlockSpec((_H, _H), full),
            pl.BlockSpec((1, _H), full),
            pl.BlockSpec((_H, _D), full),
            pl.BlockSpec((1, _D), full),
            pl.BlockSpec((1, 2 * _D), full),
            pl.BlockSpec((1, 1), full),
        ],
        out_specs=pl.BlockSpec((_BB, 1), row),
        out_shape=jax.ShapeDtypeStruct((_B, 1), jnp.float32),
    )(*xs, wide_W, wide_b.reshape(1, _D), dW0, db0.reshape(1, _H),
      dW1, db1.reshape(1, _H), dW2, db2.reshape(1, _H), dW3,
      db3.reshape(1, _D), tW.reshape(1, 2 * _D), tb.reshape(1, 1))


def kernel(user_ids, item_ids, sparse_features, user_table, item_table,
           sparse_tables, wide_W, wide_b, dW0, db0, dW1, db1, dW2, db2,
           dW3, db3, tW, tb):
    ui = user_ids.reshape(1, _B)
    ii = item_ids.reshape(1, _B)
    sf_t = sparse_features.T
    xs = _sc_gather_kernel()(user_table, item_table, sparse_tables,
                             ui, ii, sf_t)
    return _dense_forward(xs, wide_W, wide_b, dW0, db0,
                          dW1, db1, dW2, db2, dW3, db3, tW, tb)


# speedup vs baseline: 1.1340x; 1.0140x over previous
import functools

import jax
import jax.numpy as jnp
from jax.experimental import pallas as pl
from jax.experimental.pallas import tpu as pltpu
from jax.experimental.pallas import tpu_sc as plsc

_B = 16384
_D = 32
_F = 26
_SPARSE_V = 100000
_DIN = (_F + 2) * _D
_H = 2 * _D
_W = 128
_BB = 512
_NJ = _F + 2


def _gather_pipeline(table_hbm, idx_hbm, idx_row, out_hbm):
    def body(i_vmem, o_vmem):
        pltpu.sync_copy(table_hbm.at[i_vmem.at[0]], o_vmem)

    pltpu.emit_pipeline(
        body,
        grid=(_B // _W,),
        in_specs=[pl.BlockSpec((1, _W), lambda i, r=idx_row: (r, i))],
        out_specs=[pl.BlockSpec((_W, 128), lambda i: (i, 0))],
        core_axis_name=("c", "s"),
        dimension_semantics=(pltpu.PARALLEL,),
    )(idx_hbm, out_hbm)


@functools.cache
def _sc_gather_kernel():
    mesh = plsc.VectorSubcoreMesh(core_axis_name="c", subcore_axis_name="s")
    emb = jax.ShapeDtypeStruct((_B, 128), jnp.float32)

    @functools.partial(
        pl.kernel,
        out_type=(emb,) * _NJ,
        mesh=mesh,
    )
    def sc_gather(u_hbm, i_hbm, s_hbm, ui_hbm, ii_hbm, si_hbm, *outs):
        _gather_pipeline(u_hbm, ui_hbm, 0, outs[0])
        _gather_pipeline(i_hbm, ii_hbm, 0, outs[1])
        for f in range(_F):
            _gather_pipeline(s_hbm, si_hbm, f, outs[2 + f])

    return sc_gather


def _dense_body(*refs):
    x_refs = refs[:_NJ]
    (sel_ref, wWb_ref, wb_ref, w0b_ref, b0_ref, w1_ref, b1_ref,
     w2_ref, b2_ref, w3_ref, b3_ref, tw_ref, tb_ref, o_ref) = refs[_NJ:]
    dot = lambda a, b: jax.lax.dot_general(
        a, b, (((1,), (0,)), ((), ())), preferred_element_type=jnp.float32)
    w0b = w0b_ref[...]
    wWb = wWb_ref[...]
    lane_q = jax.lax.broadcasted_iota(jnp.int32, (_BB, 128), 1) // 32
    hacc = None
    wacc = None
    for j in range(_NJ):
        x = x_refs[j][...]
        sel = sel_ref[:, j:j + 1]
        e = jnp.where(lane_q == sel, x, 0.0)
        hj = dot(e, w0b[128 * j:128 * j + 128])
        wj = dot(e, wWb[128 * j:128 * j + 128])
        hacc = hj if hacc is None else hacc + hj
        wacc = wj if wacc is None else wacc + wj
    h = jax.nn.relu(hacc + b0_ref[...])
    h = jax.nn.relu(dot(h, w1_ref[...]) + b1_ref[...])
    h = jax.nn.relu(dot(h, w2_ref[...]) + b2_ref[...])
    deep = dot(h, w3_ref[...]) + b3_ref[...]
    wide = wacc + wb_ref[...]
    tw = tw_ref[...]
    logit = (jnp.sum(wide * tw[:, 0:_D], axis=1, keepdims=True)
             + jnp.sum(deep * tw[:, _D:], axis=1, keepdims=True)
             + tb_ref[...])
    o_ref[...] = jax.nn.sigmoid(logit)


def _dense_forward(xs, sel, wide_W, wide_b, dW0, db0, dW1, db1,
                   dW2, db2, dW3, db3, tW, tb):
    row = lambda i: (i, 0)
    full = lambda i: (0, 0)
    w0b = jnp.tile(dW0.reshape(_NJ, 1, _D, _H), (1, 4, 1, 1)).reshape(_NJ * 128, _H)
    wWb = jnp.tile(wide_W.reshape(_NJ, 1, _D, _D), (1, 4, 1, 1)).reshape(_NJ * 128, _D)
    return pl.pallas_call(
        _dense_body,
        grid=(_B // _BB,),
        in_specs=[pl.BlockSpec((_BB, 128), row)] * _NJ + [
            pl.BlockSpec((_BB, _NJ), row),
            pl.BlockSpec((_NJ * 128, _D), full),
            pl.BlockSpec((1, _D), full),
            pl.BlockSpec((_NJ * 128, _H), full),
            pl.BlockSpec((1, _H), full),
            pl.BlockSpec((_H, _H), full),
            pl.BlockSpec((1, _H), full),
            pl.BlockSpec((_H, _H), full),
            pl.BlockSpec((1, _H), full),
            pl.BlockSpec((_H, _D), full),
            pl.BlockSpec((1, _D), full),
            pl.BlockSpec((1, 2 * _D), full),
            pl.BlockSpec((1, 1), full),
        ],
        out_specs=pl.BlockSpec((_BB, 1), row),
        out_shape=jax.ShapeDtypeStruct((_B, 1), jnp.float32),
    )(*xs, sel, wWb, wide_b.reshape(1, _D), w0b, db0.reshape(1, _H),
      dW1, db1.reshape(1, _H), dW2, db2.reshape(1, _H), dW3,
      db3.reshape(1, _D), tW.reshape(1, 2 * _D), tb.reshape(1, 1))


def kernel(user_ids, item_ids, sparse_features, user_table, item_table,
           sparse_tables, wide_W, wide_b, dW0, db0, dW1, db1, dW2, db2,
           dW3, db3, tW, tb):
    u4 = user_table.reshape(-1, 128)
    i4 = item_table.reshape(-1, 128)
    s4 = sparse_tables.reshape(-1, 128)
    ui = (user_ids // 4).reshape(1, _B)
    ii = (item_ids // 4).reshape(1, _B)
    sf_t = sparse_features.T
    sp_offs = (jnp.arange(_F, dtype=jnp.int32) * (_SPARSE_V // 4))[:, None]
    si = sf_t // 4 + sp_offs
    sel = jnp.concatenate([(user_ids % 4)[:, None], (item_ids % 4)[:, None],
                           sparse_features % 4], axis=1)
    xs = _sc_gather_kernel()(u4, i4, s4, ui, ii, si)
    return _dense_forward(xs, sel, wide_W, wide_b, dW0, db0,
                          dW1, db1, dW2, db2, dW3, db3, tW, tb)


# speedup vs baseline: 1.2069x; 1.0643x over previous
import functools

import jax
import jax.numpy as jnp
from jax.experimental import pallas as pl
from jax.experimental.pallas import tpu as pltpu
from jax.experimental.pallas import tpu_sc as plsc

_B = 16384
_D = 32
_F = 26
_SPARSE_V = 100000
_DIN = (_F + 2) * _D
_H = 2 * _D
_W = 128
_BB = 512
_NJ = _F + 2


_RB = 2000


def _detile_body(x0_ref, x1_ref, x2_ref, x3_ref, o_ref):
    o_ref[...] = jnp.concatenate(
        [x0_ref[...], x1_ref[...], x2_ref[...], x3_ref[...]], axis=1)


def _detile(table):
    v = table.shape[0]
    q = v // 4
    nb = q // _RB
    specs = [pl.BlockSpec((_RB, _D), lambda i, c=c, n=nb: (n * c + i, 0))
             for c in range(4)]
    return pl.pallas_call(
        _detile_body,
        grid=(nb,),
        in_specs=[specs[0], specs[1], specs[2], specs[3]],
        out_specs=pl.BlockSpec((_RB, 128), lambda i: (i, 0)),
        out_shape=jax.ShapeDtypeStruct((q, 128), jnp.float32),
    )(table, table, table, table)


def _gather_pipeline(table_hbm, idx_hbm, idx_row, out_hbm):
    def body(i_vmem, o_vmem):
        pltpu.sync_copy(table_hbm.at[i_vmem.at[0]], o_vmem)

    pltpu.emit_pipeline(
        body,
        grid=(_B // _W,),
        in_specs=[pl.BlockSpec((1, _W), lambda i, r=idx_row: (r, i))],
        out_specs=[pl.BlockSpec((_W, 128), lambda i: (i, 0))],
        core_axis_name=("c", "s"),
        dimension_semantics=(pltpu.PARALLEL,),
    )(idx_hbm, out_hbm)


@functools.cache
def _sc_gather_kernel():
    mesh = plsc.VectorSubcoreMesh(core_axis_name="c", subcore_axis_name="s")
    emb = jax.ShapeDtypeStruct((_B, 128), jnp.float32)

    @functools.partial(
        pl.kernel,
        out_type=(emb,) * _NJ,
        mesh=mesh,
    )
    def sc_gather(u_hbm, i_hbm, s_hbm, ui_hbm, ii_hbm, si_hbm, *outs):
        _gather_pipeline(u_hbm, ui_hbm, 0, outs[0])
        _gather_pipeline(i_hbm, ii_hbm, 0, outs[1])
        for f in range(_F):
            _gather_pipeline(s_hbm, si_hbm, f, outs[2 + f])

    return sc_gather


def _dense_body(*refs):
    x_refs = refs[:_NJ]
    (sel_ref, wWb_ref, wb_ref, w0b_ref, b0_ref, w1_ref, b1_ref,
     w2_ref, b2_ref, w3_ref, b3_ref, tw_ref, tb_ref, o_ref) = refs[_NJ:]
    dot = lambda a, b: jax.lax.dot_general(
        a, b, (((1,), (0,)), ((), ())), preferred_element_type=jnp.float32)
    w0b = w0b_ref[...]
    wWb = wWb_ref[...]
    lane_q = jax.lax.broadcasted_iota(jnp.int32, (_BB, 128), 1) // 32
    hacc = None
    wacc = None
    for j in range(_NJ):
        x = x_refs[j][...]
        sel = sel_ref[:, j:j + 1]
        e = jnp.where(lane_q == sel, x, 0.0)
        hj = dot(e, w0b[128 * j:128 * j + 128])
        wj = dot(e, wWb[128 * j:128 * j + 128])
        hacc = hj if hacc is None else hacc + hj
        wacc = wj if wacc is None else wacc + wj
    h = jax.nn.relu(hacc + b0_ref[...])
    h = jax.nn.relu(dot(h, w1_ref[...]) + b1_ref[...])
    h = jax.nn.relu(dot(h, w2_ref[...]) + b2_ref[...])
    deep = dot(h, w3_ref[...]) + b3_ref[...]
    wide = wacc + wb_ref[...]
    tw = tw_ref[...]
    logit = (jnp.sum(wide * tw[:, 0:_D], axis=1, keepdims=True)
             + jnp.sum(deep * tw[:, _D:], axis=1, keepdims=True)
             + tb_ref[...])
    o_ref[...] = jax.nn.sigmoid(logit)


def _dense_forward(xs, sel, wide_W, wide_b, dW0, db0, dW1, db1,
                   dW2, db2, dW3, db3, tW, tb):
    row = lambda i: (i, 0)
    full = lambda i: (0, 0)
    w0b = jnp.tile(dW0.reshape(_NJ, 1, _D, _H), (1, 4, 1, 1)).reshape(_NJ * 128, _H)
    wWb = jnp.tile(wide_W.reshape(_NJ, 1, _D, _D), (1, 4, 1, 1)).reshape(_NJ * 128, _D)
    return pl.pallas_call(
        _dense_body,
        grid=(_B // _BB,),
        in_specs=[pl.BlockSpec((_BB, 128), row)] * _NJ + [
            pl.BlockSpec((_BB, _NJ), row),
            pl.BlockSpec((_NJ * 128, _D), full),
            pl.BlockSpec((1, _D), full),
            pl.BlockSpec((_NJ * 128, _H), full),
            pl.BlockSpec((1, _H), full),
            pl.BlockSpec((_H, _H), full),
            pl.BlockSpec((1, _H), full),
            pl.BlockSpec((_H, _H), full),
            pl.BlockSpec((1, _H), full),
            pl.BlockSpec((_H, _D), full),
            pl.BlockSpec((1, _D), full),
            pl.BlockSpec((1, 2 * _D), full),
            pl.BlockSpec((1, 1), full),
        ],
        out_specs=pl.BlockSpec((_BB, 1), row),
        out_shape=jax.ShapeDtypeStruct((_B, 1), jnp.float32),
    )(*xs, sel, wWb, wide_b.reshape(1, _D), w0b, db0.reshape(1, _H),
      dW1, db1.reshape(1, _H), dW2, db2.reshape(1, _H), dW3,
      db3.reshape(1, _D), tW.reshape(1, 2 * _D), tb.reshape(1, 1))


def kernel(user_ids, item_ids, sparse_features, user_table, item_table,
           sparse_tables, wide_W, wide_b, dW0, db0, dW1, db1, dW2, db2,
           dW3, db3, tW, tb):
    u4 = _detile(user_table)
    i4 = _detile(item_table)
    s4 = _detile(sparse_tables.reshape(_F * _SPARSE_V, _D))
    qu = (_SPARSE_V * 10) // 4
    qs = (_F * _SPARSE_V) // 4
    ui = (user_ids % qu).reshape(1, _B)
    ii = (item_ids % qu).reshape(1, _B)
    sf_t = sparse_features.T
    offs_col = (jnp.arange(_F, dtype=jnp.int32) * _SPARSE_V)[:, None]
    si = (sf_t + offs_col) % qs
    offs_row = (jnp.arange(_F, dtype=jnp.int32) * _SPARSE_V)[None, :]
    sel = jnp.concatenate([(user_ids // qu)[:, None], (item_ids // qu)[:, None],
                           (sparse_features + offs_row) // qs], axis=1)
    xs = _sc_gather_kernel()(u4, i4, s4, ui, ii, si)
    return _dense_forward(xs, sel, wide_W, wide_b, dW0, db0,
                          dW1, db1, dW2, db2, dW3, db3, tW, tb)
